# baseline (speedup 1.0000x reference)
import functools

import jax
import jax.numpy as jnp
from jax import lax
from jax.experimental import pallas as pl
from jax.experimental.pallas import tpu as pltpu
from jax.experimental.pallas import tpu_sc as plsc

N = 10000
E = 320000
D = 128
H = 128
C = 10
G = 128
BN_EPS = 1e-5

NC = 2
NS = 16
NW = NC * NS
K = 128
C0 = 56
C1 = 104
CMAX = 104
ROWS_PAD = 2608
N_PAD = 10240
RPT = N_PAD // NS
RCH = RPT // K
BM = 512
NB = N_PAD // BM


@functools.cache
def _make_sc_agg():
    mesh = plsc.VectorSubcoreMesh(
        core_axis_name="c", subcore_axis_name="s",
        num_cores=NC, num_subcores=NS)

    @functools.partial(
        pl.kernel,
        out_type=jax.ShapeDtypeStruct((NC * N_PAD, D), jnp.float32),
        mesh=mesh,
        scratch_types=[
            pltpu.VMEM((CMAX, K), jnp.int32),
            pltpu.VMEM((CMAX, K), jnp.int32),
            pltpu.VMEM((K, D), jnp.float32),
            pltpu.VMEM_SHARED((N_PAD, D), jnp.float32),
            pltpu.SemaphoreType.DMA,
        ],
    )
    def _sc_agg(x_hbm, src_hbm, dst_hbm, zeros_hbm, out_hbm,
                src_v, dst_v, rows_v, acc, sem):
        cid = lax.axis_index("c")
        sid = lax.axis_index("s")
        row0 = sid * RPT
        cnt = jnp.where(cid == 0, C0, C1)
        base = jnp.where(cid == 0, sid * C0, NS * C0 + sid * C1)

        pltpu.sync_copy(zeros_hbm, rows_v)

        def zbody(r, carry):
            pltpu.sync_copy(rows_v, acc.at[pl.ds(row0 + r * K, K)])
            return carry

        lax.fori_loop(0, RCH, zbody, 0)

        pltpu.sync_copy(src_hbm.at[pl.ds(base, CMAX)], src_v)
        pltpu.sync_copy(dst_hbm.at[pl.ds(base, CMAX)], dst_v)
        plsc.subcore_barrier()

        def body(c, carry):
            pltpu.async_copy(x_hbm.at[src_v.at[c]], rows_v, sem).wait()
            pltpu.sync_copy(rows_v, acc.at[dst_v.at[c]], add=True)
            return carry

        lax.fori_loop(0, cnt, body, 0)
        plsc.subcore_barrier()

        def wbody(r, carry):
            pltpu.sync_copy(acc.at[pl.ds(row0 + r * K, K)], rows_v)
            pltpu.sync_copy(
                rows_v, out_hbm.at[pl.ds(cid * N_PAD + row0 + r * K, K)])
            return carry

        lax.fori_loop(0, RCH, wbody, 0)

    return _sc_agg


def _mlp_body(x_ref, a_ref, w1_ref, b1_ref, w2_ref, b2_ref, o_ref):
    h = x_ref[...] + a_ref[0] + a_ref[1]
    t = jnp.dot(h, w1_ref[...], preferred_element_type=jnp.float32)
    t = jnp.maximum(t + b1_ref[...], 0.0)
    o = jnp.dot(t, w2_ref[...], preferred_element_type=jnp.float32)
    o_ref[...] = jnp.maximum(o + b2_ref[...], 0.0)


_mlp_call = pl.pallas_call(
    _mlp_body,
    grid=(NB,),
    in_specs=[
        pl.BlockSpec((BM, D), lambda i: (i, 0)),
        pl.BlockSpec((NC, BM, D), lambda i: (0, i, 0)),
        pl.BlockSpec((D, H), lambda i: (0, 0)),
        pl.BlockSpec((1, H), lambda i: (0, 0)),
        pl.BlockSpec((H, H), lambda i: (0, 0)),
        pl.BlockSpec((1, H), lambda i: (0, 0)),
    ],
    out_specs=pl.BlockSpec((BM, H), lambda i: (i, 0)),
    out_shape=jax.ShapeDtypeStruct((N_PAD, H), jnp.float32),
)


def _pool_body(h_ref, b_ref, l1w_ref, l1b_ref, l2w_ref, l2b_ref, o_ref, acc):
    i = pl.program_id(0)

    @pl.when(i == 0)
    def _():
        acc[...] = jnp.zeros((G, H), jnp.float32)

    b = b_ref[0, 0, :]
    gids = lax.broadcasted_iota(jnp.int32, (G, BM), 0)
    onehot = (gids == b[None, :]).astype(jnp.float32)
    acc[...] += jnp.dot(onehot, h_ref[...], preferred_element_type=jnp.float32)

    @pl.when(i == NB - 1)
    def _():
        t = jnp.dot(acc[...], l1w_ref[...], preferred_element_type=jnp.float32)
        t = jnp.maximum(t + l1b_ref[...], 0.0)
        o_ref[...] = jnp.dot(t, l2w_ref[...],
                             preferred_element_type=jnp.float32) + l2b_ref[...]


_pool_call = pl.pallas_call(
    _pool_body,
    grid=(NB,),
    in_specs=[
        pl.BlockSpec((BM, H), lambda i: (i, 0)),
        pl.BlockSpec((1, 1, BM), lambda i: (i, 0, 0)),
        pl.BlockSpec((H, H), lambda i: (0, 0)),
        pl.BlockSpec((1, H), lambda i: (0, 0)),
        pl.BlockSpec((H, H), lambda i: (0, 0)),
        pl.BlockSpec((1, H), lambda i: (0, 0)),
    ],
    out_specs=pl.BlockSpec((G, H), lambda i: (0, 0)),
    out_shape=jax.ShapeDtypeStruct((G, H), jnp.float32),
    scratch_shapes=[pltpu.VMEM((G, H), jnp.float32)],
)


def kernel(x, edge_index, batch,
           c1_w1, c1_b1, c1_g, c1_be, c1_w2, c1_b2,
           c2_w1, c2_b1, c2_g, c2_be, c2_w2, c2_b2,
           c3_w1, c3_b1, c3_g, c3_be, c3_w2, c3_b2,
           l1_w, l1_b, l2_w, l2_b):
    src = jnp.concatenate(
        [edge_index[0],
         jnp.zeros((ROWS_PAD * K - E,), jnp.int32)]).reshape(ROWS_PAD, K)
    dst = jnp.concatenate(
        [edge_index[1],
         jnp.full((ROWS_PAD * K - E,), N_PAD - 1,
                  jnp.int32)]).reshape(ROWS_PAD, K)
    batch_p = jnp.concatenate(
        [batch, jnp.full((N_PAD - N,), G, jnp.int32)]).reshape(NB, 1, BM)
    zeros = jnp.zeros((K, D), jnp.float32)

    h = jnp.zeros((N_PAD, D), jnp.float32).at[:N].set(x)

    layers = [
        (c1_w1, c1_b1, c1_g, c1_be, c1_w2, c1_b2),
        (c2_w1, c2_b1, c2_g, c2_be, c2_w2, c2_b2),
        (c3_w1, c3_b1, c3_g, c3_be, c3_w2, c3_b2),
    ]
    sc_agg = _make_sc_agg()
    for w1, b1, g, be, w2, b2 in layers:
        s = g / jnp.sqrt(1.0 + BN_EPS)
        w1f = w1 * s[None, :]
        b1f = (b1 * s + be).reshape(1, H)
        agg = sc_agg(h, src, dst, zeros)
        h = _mlp_call(h, agg.reshape(NC, N_PAD, D), w1f, b1f, w2,
                      b2.reshape(1, H))

    l2_wp = jnp.zeros((H, H), jnp.float32).at[:, :C].set(l2_w)
    l2_bp = jnp.zeros((1, H), jnp.float32).at[0, :C].set(l2_b)
    out = _pool_call(h, batch_p, l1_w, l1_b.reshape(1, H), l2_wp, l2_bp)
    return out[:, :C]


# speedup vs baseline: 1.0636x; 1.0636x over previous
import functools

import jax
import jax.numpy as jnp
from jax import lax
from jax.experimental import pallas as pl
from jax.experimental.pallas import tpu as pltpu
from jax.experimental.pallas import tpu_sc as plsc

N = 10000
E = 320000
D = 128
H = 128
C = 10
G = 128
BN_EPS = 1e-5

NC = 2
NS = 16
NW = NC * NS
K = 128
C0 = 80
C1 = 80
CMAX = 104
ROWS_PAD = 2608
N_PAD = 10240
RPT = N_PAD // NS
RCH = RPT // K
BM = 512
NB = N_PAD // BM


@functools.cache
def _make_sc_agg():
    mesh = plsc.VectorSubcoreMesh(
        core_axis_name="c", subcore_axis_name="s",
        num_cores=NC, num_subcores=NS)

    @functools.partial(
        pl.kernel,
        out_type=jax.ShapeDtypeStruct((NC * N_PAD, D), jnp.float32),
        mesh=mesh,
        scratch_types=[
            pltpu.VMEM((CMAX, K), jnp.int32),
            pltpu.VMEM((CMAX, K), jnp.int32),
            pltpu.VMEM((K, D), jnp.float32),
            pltpu.VMEM_SHARED((N_PAD, D), jnp.float32),
            pltpu.SemaphoreType.DMA,
        ],
    )
    def _sc_agg(x_hbm, src_hbm, dst_hbm, zeros_hbm, out_hbm,
                src_v, dst_v, rows_v, acc, sem):
        cid = lax.axis_index("c")
        sid = lax.axis_index("s")
        row0 = sid * RPT
        cnt = jnp.where(cid == 0, C0, C1)
        base = jnp.where(cid == 0, sid * C0, NS * C0 + sid * C1)

        pltpu.sync_copy(zeros_hbm, rows_v)

        def zbody(r, carry):
            pltpu.sync_copy(rows_v, acc.at[pl.ds(row0 + r * K, K)])
            return carry

        lax.fori_loop(0, RCH, zbody, 0)

        pltpu.sync_copy(src_hbm.at[pl.ds(base, CMAX)], src_v)
        pltpu.sync_copy(dst_hbm.at[pl.ds(base, CMAX)], dst_v)
        plsc.subcore_barrier()

        def body(c, carry):
            pltpu.async_copy(x_hbm.at[src_v.at[c]], rows_v, sem).wait()
            pltpu.sync_copy(rows_v, acc.at[dst_v.at[c]], add=True)
            return carry

        lax.fori_loop(0, cnt, body, 0)
        plsc.subcore_barrier()

        def wbody(r, carry):
            pltpu.sync_copy(acc.at[pl.ds(row0 + r * K, K)], rows_v)
            pltpu.sync_copy(
                rows_v, out_hbm.at[pl.ds(cid * N_PAD + row0 + r * K, K)])
            return carry

        lax.fori_loop(0, RCH, wbody, 0)

    return _sc_agg


def _mlp_body(x_ref, a_ref, w1_ref, b1_ref, w2_ref, b2_ref, o_ref):
    h = x_ref[...] + a_ref[0] + a_ref[1]
    t = jnp.dot(h, w1_ref[...], preferred_element_type=jnp.float32)
    t = jnp.maximum(t + b1_ref[...], 0.0)
    o = jnp.dot(t, w2_ref[...], preferred_element_type=jnp.float32)
    o_ref[...] = jnp.maximum(o + b2_ref[...], 0.0)


_mlp_call = pl.pallas_call(
    _mlp_body,
    grid=(NB,),
    in_specs=[
        pl.BlockSpec((BM, D), lambda i: (i, 0)),
        pl.BlockSpec((NC, BM, D), lambda i: (0, i, 0)),
        pl.BlockSpec((D, H), lambda i: (0, 0)),
        pl.BlockSpec((1, H), lambda i: (0, 0)),
        pl.BlockSpec((H, H), lambda i: (0, 0)),
        pl.BlockSpec((1, H), lambda i: (0, 0)),
    ],
    out_specs=pl.BlockSpec((BM, H), lambda i: (i, 0)),
    out_shape=jax.ShapeDtypeStruct((N_PAD, H), jnp.float32),
)


def _pool_body(h_ref, b_ref, l1w_ref, l1b_ref, l2w_ref, l2b_ref, o_ref, acc):
    i = pl.program_id(0)

    @pl.when(i == 0)
    def _():
        acc[...] = jnp.zeros((G, H), jnp.float32)

    b = b_ref[0, 0, :]
    gids = lax.broadcasted_iota(jnp.int32, (G, BM), 0)
    onehot = (gids == b[None, :]).astype(jnp.float32)
    acc[...] += jnp.dot(onehot, h_ref[...], preferred_element_type=jnp.float32)

    @pl.when(i == NB - 1)
    def _():
        t = jnp.dot(acc[...], l1w_ref[...], preferred_element_type=jnp.float32)
        t = jnp.maximum(t + l1b_ref[...], 0.0)
        o_ref[...] = jnp.dot(t, l2w_ref[...],
                             preferred_element_type=jnp.float32) + l2b_ref[...]


_pool_call = pl.pallas_call(
    _pool_body,
    grid=(NB,),
    in_specs=[
        pl.BlockSpec((BM, H), lambda i: (i, 0)),
        pl.BlockSpec((1, 1, BM), lambda i: (i, 0, 0)),
        pl.BlockSpec((H, H), lambda i: (0, 0)),
        pl.BlockSpec((1, H), lambda i: (0, 0)),
        pl.BlockSpec((H, H), lambda i: (0, 0)),
        pl.BlockSpec((1, H), lambda i: (0, 0)),
    ],
    out_specs=pl.BlockSpec((G, H), lambda i: (0, 0)),
    out_shape=jax.ShapeDtypeStruct((G, H), jnp.float32),
    scratch_shapes=[pltpu.VMEM((G, H), jnp.float32)],
)


def kernel(x, edge_index, batch,
           c1_w1, c1_b1, c1_g, c1_be, c1_w2, c1_b2,
           c2_w1, c2_b1, c2_g, c2_be, c2_w2, c2_b2,
           c3_w1, c3_b1, c3_g, c3_be, c3_w2, c3_b2,
           l1_w, l1_b, l2_w, l2_b):
    src = jnp.concatenate(
        [edge_index[0],
         jnp.zeros((ROWS_PAD * K - E,), jnp.int32)]).reshape(ROWS_PAD, K)
    dst = jnp.concatenate(
        [edge_index[1],
         jnp.full((ROWS_PAD * K - E,), N_PAD - 1,
                  jnp.int32)]).reshape(ROWS_PAD, K)
    batch_p = jnp.concatenate(
        [batch, jnp.full((N_PAD - N,), G, jnp.int32)]).reshape(NB, 1, BM)
    zeros = jnp.zeros((K, D), jnp.float32)

    h = jnp.zeros((N_PAD, D), jnp.float32).at[:N].set(x)

    layers = [
        (c1_w1, c1_b1, c1_g, c1_be, c1_w2, c1_b2),
        (c2_w1, c2_b1, c2_g, c2_be, c2_w2, c2_b2),
        (c3_w1, c3_b1, c3_g, c3_be, c3_w2, c3_b2),
    ]
    sc_agg = _make_sc_agg()
    for w1, b1, g, be, w2, b2 in layers:
        s = g / jnp.sqrt(1.0 + BN_EPS)
        w1f = w1 * s[None, :]
        b1f = (b1 * s + be).reshape(1, H)
        agg = sc_agg(h, src, dst, zeros)
        h = _mlp_call(h, agg.reshape(NC, N_PAD, D), w1f, b1f, w2,
                      b2.reshape(1, H))

    l2_wp = jnp.zeros((H, H), jnp.float32).at[:, :C].set(l2_w)
    l2_bp = jnp.zeros((1, H), jnp.float32).at[0, :C].set(l2_b)
    out = _pool_call(h, batch_p, l1_w, l1_b.reshape(1, H), l2_wp, l2_bp)
    return out[:, :C]


# speedup vs baseline: 1.1587x; 1.0895x over previous
import functools

import jax
import jax.numpy as jnp
from jax import lax
from jax.experimental import pallas as pl
from jax.experimental.pallas import tpu as pltpu
from jax.experimental.pallas import tpu_sc as plsc

N = 10000
E = 320000
D = 128
H = 128
C = 10
G = 128
BN_EPS = 1e-5

NC = 2
NS = 16
NW = NC * NS
K = 128
C0 = 104
C1 = 56
CMAX = 104
ROWS_PAD = 2608
N_PAD = 10240
RPT = N_PAD // NS
RCH = RPT // K
BM = 512
NB = N_PAD // BM


@functools.cache
def _make_sc_agg():
    mesh = plsc.VectorSubcoreMesh(
        core_axis_name="c", subcore_axis_name="s",
        num_cores=NC, num_subcores=NS)

    @functools.partial(
        pl.kernel,
        out_type=jax.ShapeDtypeStruct((NC * N_PAD, D), jnp.float32),
        mesh=mesh,
        scratch_types=[
            pltpu.VMEM((CMAX, K), jnp.int32),
            pltpu.VMEM((CMAX, K), jnp.int32),
            pltpu.VMEM((K, D), jnp.float32),
            pltpu.VMEM_SHARED((N_PAD, D), jnp.float32),
            pltpu.SemaphoreType.DMA,
        ],
    )
    def _sc_agg(x_hbm, src_hbm, dst_hbm, zeros_hbm, out_hbm,
                src_v, dst_v, rows_v, acc, sem):
        cid = lax.axis_index("c")
        sid = lax.axis_index("s")
        row0 = sid * RPT
        base = jnp.where(cid == 0, sid * C0, NS * C0 + sid * C1)

        pltpu.sync_copy(zeros_hbm, rows_v)

        def zbody(r, carry):
            pltpu.sync_copy(rows_v, acc.at[pl.ds(row0 + r * K, K)])
            return carry

        lax.fori_loop(0, RCH, zbody, 0)

        pltpu.sync_copy(src_hbm.at[pl.ds(base, CMAX)], src_v)
        pltpu.sync_copy(dst_hbm.at[pl.ds(base, CMAX)], dst_v)
        plsc.subcore_barrier()

        def body(c, carry):
            pltpu.async_copy(x_hbm.at[src_v.at[c]], rows_v, sem).wait()
            pltpu.sync_copy(rows_v, acc.at[dst_v.at[c]], add=True)
            return carry

        @pl.when(cid == 0)
        def _():
            lax.fori_loop(0, C0, body, 0)

        @pl.when(cid == 1)
        def _():
            lax.fori_loop(0, C1, body, 0)

        plsc.subcore_barrier()

        def wbody(r, carry):
            pltpu.sync_copy(acc.at[pl.ds(row0 + r * K, K)], rows_v)
            pltpu.sync_copy(
                rows_v, out_hbm.at[pl.ds(cid * N_PAD + row0 + r * K, K)])
            return carry

        lax.fori_loop(0, RCH, wbody, 0)

    return _sc_agg


def _mlp_body(x_ref, a_ref, w1_ref, b1_ref, w2_ref, b2_ref, o_ref):
    h = x_ref[...] + a_ref[0] + a_ref[1]
    t = jnp.dot(h, w1_ref[...], preferred_element_type=jnp.float32)
    t = jnp.maximum(t + b1_ref[...], 0.0)
    o = jnp.dot(t, w2_ref[...], preferred_element_type=jnp.float32)
    o_ref[...] = jnp.maximum(o + b2_ref[...], 0.0)


_mlp_call = pl.pallas_call(
    _mlp_body,
    grid=(NB,),
    in_specs=[
        pl.BlockSpec((BM, D), lambda i: (i, 0)),
        pl.BlockSpec((NC, BM, D), lambda i: (0, i, 0)),
        pl.BlockSpec((D, H), lambda i: (0, 0)),
        pl.BlockSpec((1, H), lambda i: (0, 0)),
        pl.BlockSpec((H, H), lambda i: (0, 0)),
        pl.BlockSpec((1, H), lambda i: (0, 0)),
    ],
    out_specs=pl.BlockSpec((BM, H), lambda i: (i, 0)),
    out_shape=jax.ShapeDtypeStruct((N_PAD, H), jnp.float32),
)


def _pool_body(h_ref, b_ref, l1w_ref, l1b_ref, l2w_ref, l2b_ref, o_ref, acc):
    i = pl.program_id(0)

    @pl.when(i == 0)
    def _():
        acc[...] = jnp.zeros((G, H), jnp.float32)

    b = b_ref[0, 0, :]
    gids = lax.broadcasted_iota(jnp.int32, (G, BM), 0)
    onehot = (gids == b[None, :]).astype(jnp.float32)
    acc[...] += jnp.dot(onehot, h_ref[...], preferred_element_type=jnp.float32)

    @pl.when(i == NB - 1)
    def _():
        t = jnp.dot(acc[...], l1w_ref[...], preferred_element_type=jnp.float32)
        t = jnp.maximum(t + l1b_ref[...], 0.0)
        o_ref[...] = jnp.dot(t, l2w_ref[...],
                             preferred_element_type=jnp.float32) + l2b_ref[...]


_pool_call = pl.pallas_call(
    _pool_body,
    grid=(NB,),
    in_specs=[
        pl.BlockSpec((BM, H), lambda i: (i, 0)),
        pl.BlockSpec((1, 1, BM), lambda i: (i, 0, 0)),
        pl.BlockSpec((H, H), lambda i: (0, 0)),
        pl.BlockSpec((1, H), lambda i: (0, 0)),
        pl.BlockSpec((H, H), lambda i: (0, 0)),
        pl.BlockSpec((1, H), lambda i: (0, 0)),
    ],
    out_specs=pl.BlockSpec((G, H), lambda i: (0, 0)),
    out_shape=jax.ShapeDtypeStruct((G, H), jnp.float32),
    scratch_shapes=[pltpu.VMEM((G, H), jnp.float32)],
)


def kernel(x, edge_index, batch,
           c1_w1, c1_b1, c1_g, c1_be, c1_w2, c1_b2,
           c2_w1, c2_b1, c2_g, c2_be, c2_w2, c2_b2,
           c3_w1, c3_b1, c3_g, c3_be, c3_w2, c3_b2,
           l1_w, l1_b, l2_w, l2_b):
    src = jnp.concatenate(
        [edge_index[0],
         jnp.zeros((ROWS_PAD * K - E,), jnp.int32)]).reshape(ROWS_PAD, K)
    dst = jnp.concatenate(
        [edge_index[1],
         jnp.full((ROWS_PAD * K - E,), N_PAD - 1,
                  jnp.int32)]).reshape(ROWS_PAD, K)
    batch_p = jnp.concatenate(
        [batch, jnp.full((N_PAD - N,), G, jnp.int32)]).reshape(NB, 1, BM)
    zeros = jnp.zeros((K, D), jnp.float32)

    h = jnp.zeros((N_PAD, D), jnp.float32).at[:N].set(x)

    layers = [
        (c1_w1, c1_b1, c1_g, c1_be, c1_w2, c1_b2),
        (c2_w1, c2_b1, c2_g, c2_be, c2_w2, c2_b2),
        (c3_w1, c3_b1, c3_g, c3_be, c3_w2, c3_b2),
    ]
    sc_agg = _make_sc_agg()
    for w1, b1, g, be, w2, b2 in layers:
        s = g / jnp.sqrt(1.0 + BN_EPS)
        w1f = w1 * s[None, :]
        b1f = (b1 * s + be).reshape(1, H)
        agg = sc_agg(h, src, dst, zeros)
        h = _mlp_call(h, agg.reshape(NC, N_PAD, D), w1f, b1f, w2,
                      b2.reshape(1, H))

    l2_wp = jnp.zeros((H, H), jnp.float32).at[:, :C].set(l2_w)
    l2_bp = jnp.zeros((1, H), jnp.float32).at[0, :C].set(l2_b)
    out = _pool_call(h, batch_p, l1_w, l1_b.reshape(1, H), l2_wp, l2_bp)
    return out[:, :C]


# speedup vs baseline: 1.4757x; 1.2735x over previous
import functools

import jax
import jax.numpy as jnp
from jax import lax
from jax.experimental import pallas as pl
from jax.experimental.pallas import tpu as pltpu
from jax.experimental.pallas import tpu_sc as plsc

N = 10000
E = 320000
D = 128
H = 128
C = 10
G = 128
BN_EPS = 1e-5

NC = 2
NS = 16
NW = NC * NS
K = 128
CH = (E + NW * K - 1) // (NW * K)
E_PAD = NW * CH * K
N_PAD = 10240
RPT = N_PAD // NS
RCH = RPT // K
BM = 512
NB = N_PAD // BM


@functools.cache
def _make_sc_agg():
    mesh = plsc.VectorSubcoreMesh(
        core_axis_name="c", subcore_axis_name="s",
        num_cores=NC, num_subcores=NS)

    @functools.partial(
        pl.kernel,
        out_type=jax.ShapeDtypeStruct((NC * N_PAD, D), jnp.float32),
        mesh=mesh,
        scratch_types=[
            pltpu.VMEM((CH, K), jnp.int32),
            pltpu.VMEM((CH, K), jnp.int32),
            pltpu.VMEM((K, D), jnp.float32),
            pltpu.VMEM_SHARED((N_PAD, D), jnp.float32),
            pltpu.SemaphoreType.DMA,
        ],
    )
    def _sc_agg(x_hbm, src_hbm, dst_hbm, zeros_hbm, out_hbm,
                src_v, dst_v, rows_v, acc, sem):
        cid = lax.axis_index("c")
        sid = lax.axis_index("s")
        w = cid * NS + sid
        row0 = sid * RPT

        pltpu.sync_copy(zeros_hbm, rows_v)

        def zbody(r, carry):
            pltpu.sync_copy(rows_v, acc.at[pl.ds(row0 + r * K, K)])
            return carry

        lax.fori_loop(0, RCH, zbody, 0)

        pltpu.sync_copy(src_hbm.at[w], src_v)
        pltpu.sync_copy(dst_hbm.at[w], dst_v)
        plsc.subcore_barrier()

        def body(c, carry):
            pltpu.async_copy(x_hbm.at[src_v.at[c]], rows_v, sem).wait()
            pltpu.sync_copy(rows_v, acc.at[dst_v.at[c]], add=True)
            return carry

        lax.fori_loop(0, CH, body, 0)
        plsc.subcore_barrier()

        def wbody(r, carry):
            pltpu.sync_copy(acc.at[pl.ds(row0 + r * K, K)], rows_v)
            pltpu.sync_copy(
                rows_v, out_hbm.at[pl.ds(cid * N_PAD + row0 + r * K, K)])
            return carry

        lax.fori_loop(0, RCH, wbody, 0)

    return _sc_agg


def _mlp_body(x_ref, a_ref, w1_ref, b1_ref, w2_ref, b2_ref, o_ref):
    h = x_ref[...] + a_ref[0] + a_ref[1]
    t = jnp.dot(h, w1_ref[...], preferred_element_type=jnp.float32)
    t = jnp.maximum(t + b1_ref[...], 0.0)
    o = jnp.dot(t, w2_ref[...], preferred_element_type=jnp.float32)
    o_ref[...] = jnp.maximum(o + b2_ref[...], 0.0)


_mlp_call = pl.pallas_call(
    _mlp_body,
    grid=(NB,),
    in_specs=[
        pl.BlockSpec((BM, D), lambda i: (i, 0)),
        pl.BlockSpec((NC, BM, D), lambda i: (0, i, 0)),
        pl.BlockSpec((D, H), lambda i: (0, 0)),
        pl.BlockSpec((1, H), lambda i: (0, 0)),
        pl.BlockSpec((H, H), lambda i: (0, 0)),
        pl.BlockSpec((1, H), lambda i: (0, 0)),
    ],
    out_specs=pl.BlockSpec((BM, H), lambda i: (i, 0)),
    out_shape=jax.ShapeDtypeStruct((N_PAD, H), jnp.float32),
)


def _pool_body(h_ref, b_ref, l1w_ref, l1b_ref, l2w_ref, l2b_ref, o_ref, acc):
    i = pl.program_id(0)

    @pl.when(i == 0)
    def _():
        acc[...] = jnp.zeros((G, H), jnp.float32)

    b = b_ref[0, 0, :]
    gids = lax.broadcasted_iota(jnp.int32, (G, BM), 0)
    onehot = (gids == b[None, :]).astype(jnp.float32)
    acc[...] += jnp.dot(onehot, h_ref[...], preferred_element_type=jnp.float32)

    @pl.when(i == NB - 1)
    def _():
        t = jnp.dot(acc[...], l1w_ref[...], preferred_element_type=jnp.float32)
        t = jnp.maximum(t + l1b_ref[...], 0.0)
        o_ref[...] = jnp.dot(t, l2w_ref[...],
                             preferred_element_type=jnp.float32) + l2b_ref[...]


_pool_call = pl.pallas_call(
    _pool_body,
    grid=(NB,),
    in_specs=[
        pl.BlockSpec((BM, H), lambda i: (i, 0)),
        pl.BlockSpec((1, 1, BM), lambda i: (i, 0, 0)),
        pl.BlockSpec((H, H), lambda i: (0, 0)),
        pl.BlockSpec((1, H), lambda i: (0, 0)),
        pl.BlockSpec((H, H), lambda i: (0, 0)),
        pl.BlockSpec((1, H), lambda i: (0, 0)),
    ],
    out_specs=pl.BlockSpec((G, H), lambda i: (0, 0)),
    out_shape=jax.ShapeDtypeStruct((G, H), jnp.float32),
    scratch_shapes=[pltpu.VMEM((G, H), jnp.float32)],
)


def kernel(x, edge_index, batch,
           c1_w1, c1_b1, c1_g, c1_be, c1_w2, c1_b2,
           c2_w1, c2_b1, c2_g, c2_be, c2_w2, c2_b2,
           c3_w1, c3_b1, c3_g, c3_be, c3_w2, c3_b2,
           l1_w, l1_b, l2_w, l2_b):
    src = jnp.concatenate(
        [edge_index[0], jnp.zeros((E_PAD - E,), jnp.int32)]).reshape(NW, CH, K)
    dst = jnp.concatenate(
        [edge_index[1],
         jnp.full((E_PAD - E,), N_PAD - 1, jnp.int32)]).reshape(NW, CH, K)
    batch_p = jnp.concatenate(
        [batch, jnp.full((N_PAD - N,), G, jnp.int32)]).reshape(NB, 1, BM)
    zeros = jnp.zeros((K, D), jnp.float32)

    h = jnp.zeros((N_PAD, D), jnp.float32).at[:N].set(x)

    layers = [
        (c1_w1, c1_b1, c1_g, c1_be, c1_w2, c1_b2),
        (c2_w1, c2_b1, c2_g, c2_be, c2_w2, c2_b2),
        (c3_w1, c3_b1, c3_g, c3_be, c3_w2, c3_b2),
    ]
    sc_agg = _make_sc_agg()
    for w1, b1, g, be, w2, b2 in layers:
        s = g / jnp.sqrt(1.0 + BN_EPS)
        w1f = w1 * s[None, :]
        b1f = (b1 * s + be).reshape(1, H)
        agg = sc_agg(h, src, dst, zeros)
        h = _mlp_call(h, agg.reshape(NC, N_PAD, D), w1f, b1f, w2,
                      b2.reshape(1, H))

    l2_wp = jnp.zeros((H, H), jnp.float32).at[:, :C].set(l2_w)
    l2_bp = jnp.zeros((1, H), jnp.float32).at[0, :C].set(l2_b)
    out = _pool_call(h, batch_p, l1_w, l1_b.reshape(1, H), l2_wp, l2_bp)
    return out[:, :C]
